# Initial kernel scaffold; baseline (speedup 1.0000x reference)
import jax
import jax.numpy as jnp
from jax.experimental import pallas as pl


def kernel(x, length, embed_table, lin_w, lin_b):
    raise NotImplementedError("write your pallas kernel here")


# speedup vs baseline: 118.5301x; 118.5301x over previous
import functools

import jax
import jax.numpy as jnp
from jax import lax
from jax.experimental import pallas as pl
from jax.experimental.pallas import tpu as pltpu
from jax.experimental.pallas import tpu_sc as plsc

B = 16384
L = 200
VOCAB = 7800
DIM = 64
VPAD = 7808
NC, NS = 2, 16
NW = NC * NS
RPW = B // NW
GROUPS = RPW // 16


def _project_body(e_ref, w_ref, o_ref):
    o_ref[...] = jnp.sum(e_ref[...] * w_ref[...], axis=1, keepdims=True)


def _project(table_pad, lin_w):
    return pl.pallas_call(
        _project_body,
        out_shape=jax.ShapeDtypeStruct((VPAD, 1), jnp.float32),
    )(table_pad, lin_w)


_MESH = plsc.VectorSubcoreMesh(core_axis_name="c", subcore_axis_name="s")


@functools.partial(
    pl.kernel,
    out_type=jax.ShapeDtypeStruct((B,), jnp.float32),
    mesh=_MESH,
    compiler_params=pltpu.CompilerParams(needs_layout_passes=False),
    scratch_types=[
        pltpu.VMEM((VPAD,), jnp.float32),
        pltpu.VMEM((RPW * L,), jnp.int32),
        pltpu.VMEM((RPW,), jnp.float32),
        pltpu.VMEM((16,), jnp.float32),
        pltpu.VMEM((RPW,), jnp.float32),
    ],
)
def _sc_pool(s_hbm, x_hbm, len_hbm, bias_hbm, out_hbm,
             s_v, x_v, len_v, bias_v, out_v):
    wid = lax.axis_index("s") * NC + lax.axis_index("c")
    base = wid * RPW
    pltpu.sync_copy(s_hbm, s_v)
    pltpu.sync_copy(x_hbm.at[pl.ds(base * L, RPW * L)], x_v)
    pltpu.sync_copy(len_hbm.at[pl.ds(base, RPW)], len_v)
    pltpu.sync_copy(bias_hbm, bias_v)

    lanes = lax.iota(jnp.int32, 16)
    bias = bias_v[...]

    def group_body(g, carry):
        idx0 = (g * 16 + lanes) * L

        def inner(_, st):
            acc, idx = st
            xv = plsc.load_gather(x_v, [idx])
            val = plsc.load_gather(s_v, [xv])
            return acc + val, idx + 1

        acc, _ = lax.fori_loop(
            0, L, inner, (jnp.zeros((16,), jnp.float32), idx0))
        z = acc / len_v[pl.ds(g * 16, 16)] + bias
        out_v[pl.ds(g * 16, 16)] = 1.0 / (1.0 + jnp.exp(-z))
        return carry

    lax.fori_loop(0, GROUPS, group_body, 0)
    pltpu.sync_copy(out_v, out_hbm.at[pl.ds(base, RPW)])


def kernel(x, length, embed_table, lin_w, lin_b):
    table_pad = jnp.pad(embed_table, ((0, VPAD - VOCAB), (0, 0)))
    s = _project(table_pad, lin_w).reshape(VPAD)
    bias16 = jnp.broadcast_to(lin_b, (16,)).astype(jnp.float32)
    y = _sc_pool(s, x.reshape(-1), length, bias16)
    return y.reshape(B, 1)


# speedup vs baseline: 168.8302x; 1.4244x over previous
import functools

import jax
import jax.numpy as jnp
from jax import lax
from jax.experimental import pallas as pl
from jax.experimental.pallas import tpu as pltpu
from jax.experimental.pallas import tpu_sc as plsc

B = 16384
L = 200
VOCAB = 7800
DIM = 64
VPAD = 7808
NC, NS = 2, 16
NW = NC * NS
RPW = B // NW
GROUPS = RPW // 16


def _project_body(e_ref, w_ref, o_ref):
    o_ref[...] = jnp.sum(e_ref[...] * w_ref[...], axis=1, keepdims=True)


def _project(table_pad, lin_w):
    return pl.pallas_call(
        _project_body,
        out_shape=jax.ShapeDtypeStruct((VPAD, 1), jnp.float32),
    )(table_pad, lin_w)


_MESH = plsc.VectorSubcoreMesh(core_axis_name="c", subcore_axis_name="s")


@functools.partial(
    pl.kernel,
    out_type=jax.ShapeDtypeStruct((B,), jnp.float32),
    mesh=_MESH,
    compiler_params=pltpu.CompilerParams(needs_layout_passes=False),
    scratch_types=[
        pltpu.VMEM((VPAD,), jnp.float32),
        pltpu.VMEM((RPW * L,), jnp.int32),
        pltpu.VMEM((RPW,), jnp.float32),
        pltpu.VMEM((16,), jnp.float32),
        pltpu.VMEM((RPW,), jnp.float32),
    ],
)
def _sc_pool(s_hbm, x_hbm, len_hbm, bias_hbm, out_hbm,
             s_v, x_v, len_v, bias_v, out_v):
    wid = lax.axis_index("s") * NC + lax.axis_index("c")
    base = wid * RPW
    pltpu.sync_copy(s_hbm, s_v)
    pltpu.sync_copy(x_hbm.at[pl.ds(base * L, RPW * L)], x_v)
    pltpu.sync_copy(len_hbm.at[pl.ds(base, RPW)], len_v)
    pltpu.sync_copy(bias_hbm, bias_v)

    lanes = lax.iota(jnp.int32, 16)
    bias = bias_v[...]
    UNROLL = 8
    zero = jnp.zeros((16,), jnp.float32)

    def group_body(g, carry):
        idx0 = (g * 16 + lanes) * L

        def inner(i, st):
            acc0, acc1, idx = st
            for k in range(UNROLL):
                xv = plsc.load_gather(x_v, [idx + k])
                val = plsc.load_gather(s_v, [xv])
                if k % 2 == 0:
                    acc0 = acc0 + val
                else:
                    acc1 = acc1 + val
            return acc0, acc1, idx + UNROLL

        a0, a1, _ = lax.fori_loop(0, L // UNROLL, inner, (zero, zero, idx0))
        z = (a0 + a1) / len_v[pl.ds(g * 16, 16)] + bias
        out_v[pl.ds(g * 16, 16)] = 1.0 / (1.0 + jnp.exp(-z))
        return carry

    lax.fori_loop(0, GROUPS, group_body, 0)
    pltpu.sync_copy(out_v, out_hbm.at[pl.ds(base, RPW)])


def kernel(x, length, embed_table, lin_w, lin_b):
    table_pad = jnp.pad(embed_table, ((0, VPAD - VOCAB), (0, 0)))
    s = _project(table_pad, lin_w).reshape(VPAD)
    bias16 = jnp.broadcast_to(lin_b, (16,)).astype(jnp.float32)
    y = _sc_pool(s, x.reshape(-1), length, bias16)
    return y.reshape(B, 1)
